# baseline (speedup 1.0000x reference)
import functools

import jax
import jax.numpy as jnp
from jax import lax
from jax.experimental import pallas as pl
from jax.experimental.pallas import tpu as pltpu
from jax.experimental.pallas import tpu_sc as plsc

NN = 10000
EE = 320000
DD = 128
GG = 64
NC = 2
NS = 16
LL = 16
NW = NC * NS
CH = 128
K0 = 61
K1 = 97
KMAX = K1
EPAD = NS * (K0 + K1) * CH
NPAD = 10240
RPS = NPAD // NS

_mesh = plsc.VectorSubcoreMesh(
    core_axis_name="c", subcore_axis_name="s", num_cores=NC, num_subcores=NS)

_f32 = jnp.float32


@functools.partial(
    pl.kernel,
    out_type=jax.ShapeDtypeStruct((NC, NPAD), _f32),
    mesh=_mesh,
    scratch_types=[
        pltpu.VMEM_SHARED((NPAD,), _f32),
        pltpu.VMEM((KMAX, CH), jnp.int32),
        pltpu.VMEM((KMAX, CH), _f32),
        pltpu.VMEM((RPS,), _f32),
    ],
)
def _deg_kernel(dst_hbm, w_hbm, out_hbm, acc, didx, wbuf, zb):
    c = lax.axis_index("c")
    s = lax.axis_index("s")
    wid = s * NC + c
    kk = jnp.where(c == 0, K0, K1)

    @pl.loop(0, RPS // LL)
    def _(i):
        zb[pl.ds(i * LL, LL)] = jnp.zeros((LL,), _f32)

    pltpu.sync_copy(zb, acc.at[pl.ds(s * RPS, RPS)])
    pltpu.sync_copy(dst_hbm.at[wid], didx)
    pltpu.sync_copy(w_hbm.at[wid], wbuf)
    plsc.subcore_barrier()

    @pl.loop(0, kk)
    def _(k):
        pltpu.sync_copy(wbuf.at[k], acc.at[didx.at[k]], add=True)

    plsc.subcore_barrier()
    pltpu.sync_copy(acc.at[pl.ds(s * RPS, RPS)],
                    out_hbm.at[c, pl.ds(s * RPS, RPS)])


@functools.partial(
    pl.kernel,
    out_type=jax.ShapeDtypeStruct((NC, NPAD, DD), _f32),
    mesh=_mesh,
    scratch_types=[
        pltpu.VMEM_SHARED((NPAD, DD), _f32),
        pltpu.VMEM((KMAX, CH), jnp.int32),
        pltpu.VMEM((KMAX, CH), jnp.int32),
        pltpu.VMEM((CH,), _f32),
        pltpu.VMEM((CH, DD), _f32),
        pltpu.SemaphoreType.DMA,
    ],
)
def _rows_kernel(hp_hbm, src_hbm, dst_hbm, w_hbm, out_hbm,
                 acc, sidx, didx, wchunk, rows, sem):
    c = lax.axis_index("c")
    s = lax.axis_index("s")
    wid = s * NC + c
    kk = jnp.where(c == 0, K0, K1)

    @pl.loop(0, CH)
    def _(i):
        for j in range(DD // LL):
            rows[i, pl.ds(j * LL, LL)] = jnp.zeros((LL,), _f32)

    @pl.loop(0, RPS // CH)
    def _(r):
        pltpu.sync_copy(rows, acc.at[pl.ds(s * RPS + r * CH, CH)])

    pltpu.sync_copy(src_hbm.at[wid], sidx)
    pltpu.sync_copy(dst_hbm.at[wid], didx)
    plsc.subcore_barrier()

    @pl.loop(0, kk)
    def _(k):
        pltpu.sync_copy(w_hbm.at[wid, k], wchunk)
        pltpu.async_copy(hp_hbm.at[sidx.at[k]], rows, sem).wait()

        @pl.loop(0, CH // LL)
        def _(g):
            wv = wchunk[pl.ds(g * LL, LL)]
            for l in range(LL):
                wsp = jnp.broadcast_to(wv[l], (LL,))
                r = g * LL + l
                for j in range(DD // LL):
                    rows[r, pl.ds(j * LL, LL)] = (
                        rows[r, pl.ds(j * LL, LL)] * wsp)

        pltpu.sync_copy(rows, acc.at[didx.at[k]], add=True)

    plsc.subcore_barrier()

    @pl.loop(0, RPS // CH)
    def _(r):
        base = s * RPS + r * CH
        pltpu.sync_copy(acc.at[pl.ds(base, CH)],
                        out_hbm.at[c, pl.ds(base, CH)])


_DN0 = (((0,), (0,)), ((), ()))
_DN1 = (((1,), (0,)), ((), ()))


def _dot(a, b, dn):
    return lax.dot_general(a, b, dn, preferred_element_type=_f32)


def _hi(a):
    return a.astype(jnp.bfloat16).astype(_f32)


def _gather_rows(st, v):
    vh = _hi(v)
    return _dot(st, vh, _DN1) + _dot(st, v - vh, _DN1)


def _dot3(a, b):
    ah, bh = _hi(a), _hi(b)
    return _dot(ah, bh, _DN1) + _dot(ah, b - bh, _DN1) + _dot(a - ah, bh, _DN1)


def _graph_norm(z, st, cnt, w, b, ms):
    mean = _dot(st, z, _DN0) / cnt
    out = z - _gather_rows(st, mean) * ms
    var = _dot(st, out * out, _DN0) / cnt
    std = jnp.sqrt(var + 1e-5)
    return w * out / _gather_rows(st, std) + b


def _onehot(batch_col):
    gids = lax.broadcasted_iota(jnp.int32, (1, GG), 1)
    st = (batch_col == gids).astype(_f32)
    ones = jnp.ones((NN, 1), _f32)
    cnt = jnp.maximum(
        _dot(st, ones, _DN0), 1.0)
    return st, cnt


def _tc_pre_body(degt_ref, x_ref, w1_ref, dinv_ref, hp_ref):
    deg = degt_ref[0:NN, 0:1] + degt_ref[0:NN, 1:2] + 1.0
    dinv = jnp.where(deg > 0, lax.rsqrt(jnp.maximum(deg, 1e-12)),
                     jnp.zeros_like(deg))
    dinv_ref[...] = dinv
    h = _dot3(x_ref[...], w1_ref[...])
    hp_ref[...] = h * dinv


def _tc_mid_body(acc_ref, hp_ref, dinv_ref, batch_ref, b1_ref,
                 gw_ref, gb_ref, gms_ref, w2_ref, hp2_ref):
    agg = acc_ref[0, 0:NN, :] + acc_ref[1, 0:NN, :] + hp_ref[...]
    z = agg * dinv_ref[...] + b1_ref[...]
    st, cnt = _onehot(batch_ref[...])
    r = jnp.maximum(
        _graph_norm(z, st, cnt, gw_ref[...], gb_ref[...], gms_ref[...]), 0.0)
    h2 = _dot3(r, w2_ref[...])
    hp2_ref[...] = h2 * dinv_ref[...]


def _tc_fin_body(acc_ref, hp_ref, dinv_ref, batch_ref, b2_ref,
                 gw_ref, gb_ref, gms_ref, out_ref):
    agg = acc_ref[0, 0:NN, :] + acc_ref[1, 0:NN, :] + hp_ref[...]
    z = agg * dinv_ref[...] + b2_ref[...]
    st, cnt = _onehot(batch_ref[...])
    out_ref[...] = jnp.maximum(
        _graph_norm(z, st, cnt, gw_ref[...], gb_ref[...], gms_ref[...]), 0.0)


_tc_pre = pl.pallas_call(
    _tc_pre_body,
    out_shape=(jax.ShapeDtypeStruct((NN, 1), _f32),
               jax.ShapeDtypeStruct((NN, DD), _f32)))

_tc_mid = pl.pallas_call(
    _tc_mid_body,
    out_shape=jax.ShapeDtypeStruct((NN, DD), _f32))

_tc_fin = pl.pallas_call(
    _tc_fin_body,
    out_shape=jax.ShapeDtypeStruct((NN, DD), _f32))


def kernel(x, edge_index, edge_weight, batch, W1, b1, W2, b2,
           gn1_w, gn1_b, gn1_ms, gn2_w, gn2_b, gn2_ms):
    pad = EPAD - EE

    def _split(flat):
        flat = jnp.pad(flat, (0, pad))
        na = NS * K0 * CH
        a = jnp.pad(flat[:na].reshape(NS, K0, CH),
                    ((0, 0), (0, KMAX - K0), (0, 0)))
        b = flat[na:].reshape(NS, K1, CH)
        return jnp.stack([a, b], axis=1).reshape(NW, KMAX, CH)

    srcp = _split(edge_index[0])
    dstp = _split(edge_index[1])
    wp = _split(edge_weight)
    batch_col = batch.reshape(NN, 1)

    degp = _deg_kernel(dstp, wp)
    dinv, hp1 = _tc_pre(degp.T, x, W1)
    acc1 = _rows_kernel(hp1, srcp, dstp, wp)
    hp2 = _tc_mid(acc1, hp1, dinv, batch_col, b1.reshape(1, DD),
                  gn1_w.reshape(1, DD), gn1_b.reshape(1, DD),
                  gn1_ms.reshape(1, DD), W2)
    acc2 = _rows_kernel(hp2, srcp, dstp, wp)
    out = _tc_fin(acc2, hp2, dinv, batch_col, b2.reshape(1, DD),
                  gn2_w.reshape(1, DD), gn2_b.reshape(1, DD),
                  gn2_ms.reshape(1, DD))
    return out


# speedup vs baseline: 1.2378x; 1.2378x over previous
import functools

import jax
import jax.numpy as jnp
from jax import lax
from jax.experimental import pallas as pl
from jax.experimental.pallas import tpu as pltpu
from jax.experimental.pallas import tpu_sc as plsc

NN = 10000
EE = 320000
DD = 128
GG = 64
NC = 2
NS = 16
LL = 16
NW = NC * NS
CH = 128
K0 = 97
K1 = 61
KMAX = K0
EPAD = NS * (K0 + K1) * CH
NPAD = 10240
RPS = NPAD // NS

_mesh = plsc.VectorSubcoreMesh(
    core_axis_name="c", subcore_axis_name="s", num_cores=NC, num_subcores=NS)

_f32 = jnp.float32


@functools.partial(
    pl.kernel,
    out_type=jax.ShapeDtypeStruct((NC, NPAD), _f32),
    mesh=_mesh,
    scratch_types=[
        pltpu.VMEM_SHARED((NPAD,), _f32),
        pltpu.VMEM((KMAX, CH), jnp.int32),
        pltpu.VMEM((KMAX, CH), _f32),
        pltpu.VMEM((RPS,), _f32),
    ],
)
def _deg_kernel(dst_hbm, w_hbm, out_hbm, acc, didx, wbuf, zb):
    c = lax.axis_index("c")
    s = lax.axis_index("s")
    wid = s * NC + c
    kk = jnp.where(c == 0, K0, K1)

    @pl.loop(0, RPS // LL)
    def _(i):
        zb[pl.ds(i * LL, LL)] = jnp.zeros((LL,), _f32)

    pltpu.sync_copy(zb, acc.at[pl.ds(s * RPS, RPS)])
    pltpu.sync_copy(dst_hbm.at[wid], didx)
    pltpu.sync_copy(w_hbm.at[wid], wbuf)
    plsc.subcore_barrier()

    @pl.loop(0, kk)
    def _(k):
        pltpu.sync_copy(wbuf.at[k], acc.at[didx.at[k]], add=True)

    plsc.subcore_barrier()
    pltpu.sync_copy(acc.at[pl.ds(s * RPS, RPS)],
                    out_hbm.at[c, pl.ds(s * RPS, RPS)])


@functools.partial(
    pl.kernel,
    out_type=jax.ShapeDtypeStruct((NC, NPAD, DD), _f32),
    mesh=_mesh,
    scratch_types=[
        pltpu.VMEM_SHARED((NPAD, DD), _f32),
        pltpu.VMEM((KMAX, CH), jnp.int32),
        pltpu.VMEM((KMAX, CH), jnp.int32),
        pltpu.VMEM((CH,), _f32),
        pltpu.VMEM((CH, DD), _f32),
        pltpu.SemaphoreType.DMA,
    ],
)
def _rows_kernel(hp_hbm, src_hbm, dst_hbm, w_hbm, out_hbm,
                 acc, sidx, didx, wchunk, rows, sem):
    c = lax.axis_index("c")
    s = lax.axis_index("s")
    wid = s * NC + c
    kk = jnp.where(c == 0, K0, K1)

    @pl.loop(0, CH)
    def _(i):
        for j in range(DD // LL):
            rows[i, pl.ds(j * LL, LL)] = jnp.zeros((LL,), _f32)

    @pl.loop(0, RPS // CH)
    def _(r):
        pltpu.sync_copy(rows, acc.at[pl.ds(s * RPS + r * CH, CH)])

    pltpu.sync_copy(src_hbm.at[wid], sidx)
    pltpu.sync_copy(dst_hbm.at[wid], didx)
    plsc.subcore_barrier()

    @pl.loop(0, kk)
    def _(k):
        pltpu.sync_copy(w_hbm.at[wid, k], wchunk)
        pltpu.async_copy(hp_hbm.at[sidx.at[k]], rows, sem).wait()

        @pl.loop(0, CH // LL)
        def _(g):
            wv = wchunk[pl.ds(g * LL, LL)]
            for l in range(LL):
                wsp = jnp.broadcast_to(wv[l], (LL,))
                r = g * LL + l
                for j in range(DD // LL):
                    rows[r, pl.ds(j * LL, LL)] = (
                        rows[r, pl.ds(j * LL, LL)] * wsp)

        pltpu.sync_copy(rows, acc.at[didx.at[k]], add=True)

    plsc.subcore_barrier()

    @pl.loop(0, RPS // CH)
    def _(r):
        base = s * RPS + r * CH
        pltpu.sync_copy(acc.at[pl.ds(base, CH)],
                        out_hbm.at[c, pl.ds(base, CH)])


_DN0 = (((0,), (0,)), ((), ()))
_DN1 = (((1,), (0,)), ((), ()))


def _dot(a, b, dn):
    return lax.dot_general(a, b, dn, preferred_element_type=_f32)


def _hi(a):
    return a.astype(jnp.bfloat16).astype(_f32)


def _gather_rows(st, v):
    vh = _hi(v)
    return _dot(st, vh, _DN1) + _dot(st, v - vh, _DN1)


def _dot3(a, b):
    ah, bh = _hi(a), _hi(b)
    return _dot(ah, bh, _DN1) + _dot(ah, b - bh, _DN1) + _dot(a - ah, bh, _DN1)


def _graph_norm(z, st, cnt, w, b, ms):
    mean = _dot(st, z, _DN0) / cnt
    out = z - _gather_rows(st, mean) * ms
    var = _dot(st, out * out, _DN0) / cnt
    std = jnp.sqrt(var + 1e-5)
    return w * out / _gather_rows(st, std) + b


def _onehot(batch_col):
    gids = lax.broadcasted_iota(jnp.int32, (1, GG), 1)
    st = (batch_col == gids).astype(_f32)
    ones = jnp.ones((NN, 1), _f32)
    cnt = jnp.maximum(
        _dot(st, ones, _DN0), 1.0)
    return st, cnt


def _tc_pre_body(degt_ref, x_ref, w1_ref, dinv_ref, hp_ref):
    deg = degt_ref[0:NN, 0:1] + degt_ref[0:NN, 1:2] + 1.0
    dinv = jnp.where(deg > 0, lax.rsqrt(jnp.maximum(deg, 1e-12)),
                     jnp.zeros_like(deg))
    dinv_ref[...] = dinv
    h = _dot3(x_ref[...], w1_ref[...])
    hp_ref[...] = h * dinv


def _tc_mid_body(acc_ref, hp_ref, dinv_ref, batch_ref, b1_ref,
                 gw_ref, gb_ref, gms_ref, w2_ref, hp2_ref):
    agg = acc_ref[0, 0:NN, :] + acc_ref[1, 0:NN, :] + hp_ref[...]
    z = agg * dinv_ref[...] + b1_ref[...]
    st, cnt = _onehot(batch_ref[...])
    r = jnp.maximum(
        _graph_norm(z, st, cnt, gw_ref[...], gb_ref[...], gms_ref[...]), 0.0)
    h2 = _dot3(r, w2_ref[...])
    hp2_ref[...] = h2 * dinv_ref[...]


def _tc_fin_body(acc_ref, hp_ref, dinv_ref, batch_ref, b2_ref,
                 gw_ref, gb_ref, gms_ref, out_ref):
    agg = acc_ref[0, 0:NN, :] + acc_ref[1, 0:NN, :] + hp_ref[...]
    z = agg * dinv_ref[...] + b2_ref[...]
    st, cnt = _onehot(batch_ref[...])
    out_ref[...] = jnp.maximum(
        _graph_norm(z, st, cnt, gw_ref[...], gb_ref[...], gms_ref[...]), 0.0)


_tc_pre = pl.pallas_call(
    _tc_pre_body,
    out_shape=(jax.ShapeDtypeStruct((NN, 1), _f32),
               jax.ShapeDtypeStruct((NN, DD), _f32)))

_tc_mid = pl.pallas_call(
    _tc_mid_body,
    out_shape=jax.ShapeDtypeStruct((NN, DD), _f32))

_tc_fin = pl.pallas_call(
    _tc_fin_body,
    out_shape=jax.ShapeDtypeStruct((NN, DD), _f32))


def kernel(x, edge_index, edge_weight, batch, W1, b1, W2, b2,
           gn1_w, gn1_b, gn1_ms, gn2_w, gn2_b, gn2_ms):
    pad = EPAD - EE

    def _split(flat):
        flat = jnp.pad(flat, (0, pad))
        na = NS * K0 * CH
        a = jnp.pad(flat[:na].reshape(NS, K0, CH),
                    ((0, 0), (0, KMAX - K0), (0, 0)))
        b = jnp.pad(flat[na:].reshape(NS, K1, CH),
                    ((0, 0), (0, KMAX - K1), (0, 0)))
        return jnp.stack([a, b], axis=1).reshape(NW, KMAX, CH)

    srcp = _split(edge_index[0])
    dstp = _split(edge_index[1])
    wp = _split(edge_weight)
    batch_col = batch.reshape(NN, 1)

    degp = _deg_kernel(dstp, wp)
    dinv, hp1 = _tc_pre(degp.T, x, W1)
    acc1 = _rows_kernel(hp1, srcp, dstp, wp)
    hp2 = _tc_mid(acc1, hp1, dinv, batch_col, b1.reshape(1, DD),
                  gn1_w.reshape(1, DD), gn1_b.reshape(1, DD),
                  gn1_ms.reshape(1, DD), W2)
    acc2 = _rows_kernel(hp2, srcp, dstp, wp)
    out = _tc_fin(acc2, hp2, dinv, batch_col, b2.reshape(1, DD),
                  gn2_w.reshape(1, DD), gn2_b.reshape(1, DD),
                  gn2_ms.reshape(1, DD))
    return out
